# baseline (speedup 1.0000x reference)
import functools

import jax
import jax.numpy as jnp
from jax import lax
from jax.experimental import pallas as pl
from jax.experimental.pallas import tpu as pltpu
from jax.experimental.pallas import tpu_sc as plsc

_NC, _NS, _L = 2, 16, 16
_NW = _NC * _NS

_MP = 512


def _sc_gather(flat, ind_p, cat_p, chw, hw):
    n = ind_p.shape[0]
    per = n // _NW
    shift = _MP.bit_length() - 1

    mesh = plsc.VectorSubcoreMesh(core_axis_name="c", subcore_axis_name="s")

    @functools.partial(
        pl.kernel,
        mesh=mesh,
        out_type=jax.ShapeDtypeStruct((n,), jnp.float32),
        scratch_types=[
            pltpu.VMEM((per,), jnp.int32),
            pltpu.VMEM((per,), jnp.int32),
            pltpu.VMEM((per,), jnp.int32),
            pltpu.VMEM((per,), jnp.float32),
            pltpu.SemaphoreType.DMA,
        ],
    )
    def gather_kernel(flat_hbm, ind_hbm, cat_hbm, out_hbm,
                      ind_v, cat_v, idx_v, val_v, sem):
        wid = lax.axis_index("s") * _NC + lax.axis_index("c")
        base = wid * per
        pltpu.sync_copy(ind_hbm.at[pl.ds(base, per)], ind_v)
        pltpu.sync_copy(cat_hbm.at[pl.ds(base, per)], cat_v)
        for k in range(per // _L):
            off = k * _L
            jv = base + off + lax.iota(jnp.int32, _L)
            bv = lax.shift_right_logical(jv, shift)
            iv = ind_v[pl.ds(off, _L)]
            cv = cat_v[pl.ds(off, _L)]
            idx_v[pl.ds(off, _L)] = bv * chw + cv * hw + iv
        pltpu.async_copy(flat_hbm.at[idx_v], val_v, sem).wait()
        pltpu.sync_copy(val_v, out_hbm.at[pl.ds(base, per)])

    return gather_kernel(flat, ind_p, cat_p)


def _dense_body(o_ref, t_ref, pk_ref, mk_ref, out_ref, acc_ref):
    i = pl.program_id(0)

    @pl.when(i == 0)
    def _init():
        acc_ref[0] = 0.0

    o = jnp.clip(o_ref[...], 0.0001, 1.0 - 0.0001)
    s = 1.0 - t_ref[...]
    s2 = s * s
    acc_ref[0] += jnp.sum(jnp.log(1.0 - o) * (o * o) * (s2 * s2))

    @pl.when(i == pl.num_programs(0) - 1)
    def _finish():
        p = jnp.clip(pk_ref[...], 0.0001, 1.0 - 0.0001)
        m = mk_ref[...]
        omp = 1.0 - p
        pos = jnp.sum(jnp.log(p) * (omp * omp) * m)
        num_pos = jnp.sum(m)
        neg = acc_ref[0]
        out_ref[0] = jnp.where(num_pos == 0.0, -neg, -(pos + neg) / num_pos)


def kernel(outx, target, ind, mask, cat):
    B, C, H, W = outx.shape
    M = ind.shape[1]
    hw = H * W
    chw = C * hw
    pad = _MP - M

    ind_p = jnp.pad(ind, ((0, 0), (0, pad))).reshape(-1)
    cat_p = jnp.pad(cat, ((0, 0), (0, pad))).reshape(-1)
    mask_p = jnp.pad(mask, ((0, 0), (0, pad)))

    peaks = _sc_gather(outx.reshape(-1), ind_p, cat_p, chw, hw)

    rows = B * C
    rb = 64
    grid = (rows // rb,)

    out2d = outx.reshape(rows, hw)
    tgt2d = target.reshape(rows, hw)
    peaks2d = peaks.reshape(B, _MP)

    res = pl.pallas_call(
        _dense_body,
        grid=grid,
        in_specs=[
            pl.BlockSpec((rb, hw), lambda i: (i, 0)),
            pl.BlockSpec((rb, hw), lambda i: (i, 0)),
            pl.BlockSpec((B, _MP), lambda i: (0, 0)),
            pl.BlockSpec((B, _MP), lambda i: (0, 0)),
        ],
        out_specs=pl.BlockSpec(memory_space=pltpu.SMEM),
        out_shape=jax.ShapeDtypeStruct((1,), jnp.float32),
        scratch_shapes=[pltpu.SMEM((1,), jnp.float32)],
    )(out2d, tgt2d, peaks2d, mask_p)
    return res[0]


# speedup vs baseline: 2.5631x; 2.5631x over previous
import functools

import jax
import jax.numpy as jnp
from jax import lax
from jax.experimental import pallas as pl
from jax.experimental.pallas import tpu as pltpu
from jax.experimental.pallas import tpu_sc as plsc

_NC, _NS, _L = 2, 16, 16
_NW = _NC * _NS

_MP = 512


def _sc_gather(flat, ind_p, cat_p, chw, hw):
    n = ind_p.shape[0]
    per = n // _NW
    shift = _MP.bit_length() - 1

    mesh = plsc.VectorSubcoreMesh(core_axis_name="c", subcore_axis_name="s")

    @functools.partial(
        pl.kernel,
        mesh=mesh,
        out_type=jax.ShapeDtypeStruct((n,), jnp.float32),
        scratch_types=[
            pltpu.VMEM((per,), jnp.int32),
            pltpu.VMEM((per,), jnp.int32),
            pltpu.VMEM((per,), jnp.int32),
            pltpu.VMEM((per,), jnp.float32),
            pltpu.SemaphoreType.DMA,
        ],
    )
    def gather_kernel(flat_hbm, ind_hbm, cat_hbm, out_hbm,
                      ind_v, cat_v, idx_v, val_v, sem):
        wid = lax.axis_index("s") * _NC + lax.axis_index("c")
        base = wid * per
        pltpu.sync_copy(ind_hbm.at[pl.ds(base, per)], ind_v)
        pltpu.sync_copy(cat_hbm.at[pl.ds(base, per)], cat_v)
        for k in range(per // _L):
            off = k * _L
            jv = base + off + lax.iota(jnp.int32, _L)
            bv = lax.shift_right_logical(jv, shift)
            iv = ind_v[pl.ds(off, _L)]
            cv = cat_v[pl.ds(off, _L)]
            idx_v[pl.ds(off, _L)] = bv * chw + cv * hw + iv
        pltpu.async_copy(flat_hbm.at[idx_v], val_v, sem).wait()
        pltpu.sync_copy(val_v, out_hbm.at[pl.ds(base, per)])

    return gather_kernel(flat, ind_p, cat_p)


def _dense_body(o_ref, t_ref, pk_ref, mk_ref, out_ref, acc_ref):
    i = pl.program_id(0)

    @pl.when(i == 0)
    def _init():
        acc_ref[0] = 0.0

    o = jnp.clip(o_ref[...], 0.0001, 1.0 - 0.0001)
    s = 1.0 - t_ref[...]
    s2 = s * s
    acc_ref[0] += jnp.sum(jnp.log(1.0 - o) * (o * o) * (s2 * s2))

    @pl.when(i == pl.num_programs(0) - 1)
    def _finish():
        p = jnp.clip(pk_ref[...], 0.0001, 1.0 - 0.0001)
        m = mk_ref[...]
        omp = 1.0 - p
        pos = jnp.sum(jnp.log(p) * (omp * omp) * m)
        num_pos = jnp.sum(m)
        neg = acc_ref[0]
        out_ref[0] = jnp.where(num_pos == 0.0, -neg, -(pos + neg) / num_pos)


def kernel(outx, target, ind, mask, cat):
    B, C, H, W = outx.shape
    M = ind.shape[1]
    hw = H * W
    chw = C * hw
    pad = _MP - M

    ind_p = jnp.pad(ind, ((0, 0), (0, pad))).reshape(-1)
    cat_p = jnp.pad(cat, ((0, 0), (0, pad))).reshape(-1)
    mask_p = jnp.pad(mask, ((0, 0), (0, pad)))

    peaks = _sc_gather(outx.reshape(-1), ind_p, cat_p, chw, hw)

    rows = B * C * H
    rb = 8192
    grid = (rows // rb,)

    out2d = outx.reshape(rows, W)
    tgt2d = target.reshape(rows, W)
    peaks2d = peaks.reshape(B, _MP)

    res = pl.pallas_call(
        _dense_body,
        grid=grid,
        in_specs=[
            pl.BlockSpec((rb, W), lambda i: (i, 0)),
            pl.BlockSpec((rb, W), lambda i: (i, 0)),
            pl.BlockSpec((B, _MP), lambda i: (0, 0)),
            pl.BlockSpec((B, _MP), lambda i: (0, 0)),
        ],
        out_specs=pl.BlockSpec(memory_space=pltpu.SMEM),
        out_shape=jax.ShapeDtypeStruct((1,), jnp.float32),
        scratch_shapes=[pltpu.SMEM((1,), jnp.float32)],
    )(out2d, tgt2d, peaks2d, mask_p)
    return res[0]
